# baseline (speedup 1.0000x reference)
import jax
import jax.numpy as jnp
from jax import lax
from jax.experimental import pallas as pl
from jax.experimental.pallas import tpu as pltpu
from jax.experimental.pallas import tpu_sc as plsc

B = 1600000
N = 100000
H = 50000

NC = 2
NS = 16
NW = NC * NS

WT = 3200
CH = 3136
CH_ROWS = CH // 16
ST = 3296
ST2_ROWS = 25
EV = B // NW
EC = 10000


def _body(src_h, dst_h, srch_h, dsth_h, out_h,
          tbl_sh, hb, stw, stc, stj, stv, stw2d, tbl_vs, tbl_vd,
          ev_s0, ev_d0, ev_o0, ev_s1, ev_d1, ev_o1,
          sem_h, sem_in, sem_out):
    c = lax.axis_index("c")
    s = lax.axis_index("s")
    wid = s * NC + c
    sid = s

    lanes = lax.broadcasted_iota(jnp.int32, (16,), 0)
    zero16 = jnp.zeros((16,), jnp.int32)

    ebase = wid * EV
    n_ch = EV // EC
    ev_s = (ev_s0, ev_s1)
    ev_d = (ev_d0, ev_d1)
    ev_o = (ev_o0, ev_o1)

    def start_in(ch, b):
        cb = pl.multiple_of(ebase + ch * EC, 8)
        d0 = pltpu.async_copy(src_h.at[pl.ds(cb, EC)], ev_s[b], sem_in[b])
        d1 = pltpu.async_copy(dst_h.at[pl.ds(cb, EC)], ev_d[b], sem_in[b])
        return (d0, d1)

    base = jnp.minimum(sid * CH, H - CH)
    skip = sid * CH - base
    pb = pl.multiple_of(jnp.maximum(base - 16, 0), 8)
    base = pl.multiple_of(base, 8)
    pend_h = []
    for hi, hist_h in enumerate((srch_h, dsth_h)):
        pend_h.append((
            pltpu.async_copy(hist_h.at[pl.ds(pb, 16)], hb[hi].at[pl.ds(0, 16)], sem_h),
            pltpu.async_copy(hist_h.at[pl.ds(base, CH)], hb[hi].at[pl.ds(16, CH)], sem_h),
        ))
    pend_in = start_in(0, 0)

    zslice = (2 * WT) // NS
    for i in range(zslice // 16):
        stv[pl.ds(i * 16, 16)] = zero16
    pltpu.sync_copy(stv.at[pl.ds(0, zslice)], tbl_sh.at[pl.ds(sid * zslice, zslice)])

    plsc.subcore_barrier()

    for hi in range(2):
        for d in pend_h[hi]:
            d.wait()
        hbuf = hb[hi]
        toff = hi * WT

        stc[pl.ds(0, 16)] = zero16
        stj[pl.ds(0, 16)] = zero16 - 1

        @plsc.parallel_loop(0, CH_ROWS, step=1, unroll=4, carry=jnp.int32(8))
        def p1(jj, off, hbuf=hbuf, toff=toff):
            o = jj * 16
            v = hbuf[pl.ds(16 + o, 16)]
            prevv = hbuf[pl.ds(15 + o, 16)]
            nxtv = hbuf[pl.ds(17 + o, 16)]
            keep = (v > prevv) | ((base == 0) & (jj == 0) & (lanes == 0))
            keep = keep & ((o + lanes) >= skip)
            bit = jnp.where(keep, jnp.int32(1) << (v & 31), 0)
            w5 = v >> 5
            isend = (w5 != (nxtv >> 5)) | (lanes == 15)
            cs = plsc.cumsum(bit)
            jjv = jnp.zeros((16,), jnp.int32) + jj
            plsc.store_compressed(stw.at[pl.ds(off, 16)], w5 + toff, mask=isend)
            plsc.store_compressed(stc.at[pl.ds(off, 16)], cs, mask=isend)
            plsc.store_compressed(stj.at[pl.ds(off, 16)], jjv, mask=isend)
            pc = plsc.all_reduce_population_count(isend)
            return off + pc[0]

        cnt = p1 - 8

        def p2(kk, _):
            q = kk * 16
            cc = stc[pl.ds(8 + q, 16)]
            cp = stc[pl.ds(7 + q, 16)]
            jc = stj[pl.ds(8 + q, 16)]
            jp = stj[pl.ds(7 + q, 16)]
            stv[pl.ds(8 + q, 16)] = cc - jnp.where(jc != jp, 0, cp)
            return _

        lax.fori_loop(0, (cnt + 15) >> 4, p2, 0)

        for k in range(8):
            stw[pl.ds(cnt + 8 + k * 16, 16)] = zero16
            stv[pl.ds(cnt + 8 + k * 16, 16)] = zero16

        for r in range(ST2_ROWS):
            @pl.when(r * 128 < cnt)
            def _(r=r):
                for k in range(8):
                    stw2d[r, pl.ds(k * 16, 16)] = stw[pl.ds(8 + r * 128 + k * 16, 16)]
                pltpu.async_copy(stv.at[pl.ds(8 + r * 128, 128)],
                                 tbl_sh.at[stw2d.at[r]], sem_h, add=True)
        for r in range(ST2_ROWS):
            @pl.when(r * 128 < cnt)
            def _(r=r):
                pltpu.make_async_copy(stv.at[pl.ds(8 + r * 128, 128)],
                                      tbl_sh.at[stw2d.at[r]], sem_h).wait()

    plsc.subcore_barrier()

    dbs = pltpu.async_copy(tbl_sh.at[pl.ds(0, WT)], tbl_vs, sem_h)
    dbd = pltpu.async_copy(tbl_sh.at[pl.ds(WT, WT)], tbl_vd, sem_h)
    dbs.wait()
    dbd.wait()

    pend_out = [None, None]
    for ch in range(n_ch):
        b = ch & 1
        for d in pend_in:
            d.wait()
        if ch + 1 < n_ch:
            pend_in = start_in(ch + 1, 1 - b)
        if pend_out[b] is not None:
            pend_out[b].wait()
        es, ed, eo = ev_s[b], ev_d[b], ev_o[b]

        @plsc.parallel_loop(0, EC, step=16, unroll=16)
        def ebody(o):
            sv = es[pl.ds(o, 16)]
            dv = ed[pl.ds(o, 16)]
            sw = plsc.load_gather(tbl_vs, [sv >> 5])
            dw = plsc.load_gather(tbl_vd, [dv >> 5])
            hit = (sw >> (sv & 31)) & (dw >> (dv & 31)) & 1
            eo[pl.ds(o, 16)] = hit.astype(jnp.float32)

        cb = pl.multiple_of(ebase + ch * EC, 8)
        pend_out[b] = pltpu.async_copy(eo, out_h.at[pl.ds(cb, EC)], sem_out[b])
    for d in pend_out:
        if d is not None:
            d.wait()


@jax.jit
def _run(src, dst, src_hist, dst_hist):
    mesh = plsc.VectorSubcoreMesh(
        core_axis_name="c", subcore_axis_name="s", num_cores=NC, num_subcores=NS)
    k = pl.kernel(
        _body,
        out_type=jax.ShapeDtypeStruct((B,), jnp.float32),
        mesh=mesh,
        compiler_params=pltpu.CompilerParams(needs_layout_passes=False),
        scratch_types=[
            pltpu.VMEM_SHARED((2 * WT,), jnp.int32),
            [pltpu.VMEM((16 + CH + 16,), jnp.int32) for _ in range(2)],
            pltpu.VMEM((ST,), jnp.int32),
            pltpu.VMEM((ST,), jnp.int32),
            pltpu.VMEM((ST,), jnp.int32),
            pltpu.VMEM((ST,), jnp.int32),
            pltpu.VMEM((ST2_ROWS, 128), jnp.int32),
            pltpu.VMEM((WT,), jnp.int32),
            pltpu.VMEM((WT,), jnp.int32),
            pltpu.VMEM((EC,), jnp.int32),
            pltpu.VMEM((EC,), jnp.int32),
            pltpu.VMEM((EC,), jnp.float32),
            pltpu.VMEM((EC,), jnp.int32),
            pltpu.VMEM((EC,), jnp.int32),
            pltpu.VMEM((EC,), jnp.float32),
            pltpu.SemaphoreType.DMA,
            [pltpu.SemaphoreType.DMA, pltpu.SemaphoreType.DMA],
            [pltpu.SemaphoreType.DMA, pltpu.SemaphoreType.DMA],
        ],
    )
    return k(src, dst, src_hist, dst_hist)


def kernel(src, dst, t, msg, src_hist, dst_hist):
    return _run(src, dst, src_hist, dst_hist)


# speedup vs baseline: 1.0841x; 1.0841x over previous
import jax
import jax.numpy as jnp
from jax import lax
from jax.experimental import pallas as pl
from jax.experimental.pallas import tpu as pltpu
from jax.experimental.pallas import tpu_sc as plsc

B = 1600000
N = 100000
H = 50000

NC = 2
NS = 16
NW = NC * NS

WT = 3200
CH = 3136
CH_ROWS = CH // 16
ST = 3296
ST2_ROWS = 25
EV = B // NW
EC = 10000


def _body(src_h, dst_h, srch_h, dsth_h, out_h,
          tbl_sh, hb, stw, stc, stj, stv, stw2d, tbl_vs, tbl_vd,
          ev_s0, ev_d0, ev_o0, ev_s1, ev_d1, ev_o1,
          sem_h, sem_in, sem_out):
    c = lax.axis_index("c")
    s = lax.axis_index("s")
    wid = s * NC + c
    sid = s

    lanes = lax.broadcasted_iota(jnp.int32, (16,), 0)
    zero16 = jnp.zeros((16,), jnp.int32)

    ebase = wid * EV
    n_ch = EV // EC
    ev_s = (ev_s0, ev_s1)
    ev_d = (ev_d0, ev_d1)
    ev_o = (ev_o0, ev_o1)

    def start_in(ch, b):
        cb = pl.multiple_of(ebase + ch * EC, 8)
        d0 = pltpu.async_copy(src_h.at[pl.ds(cb, EC)], ev_s[b], sem_in[b])
        d1 = pltpu.async_copy(dst_h.at[pl.ds(cb, EC)], ev_d[b], sem_in[b])
        return (d0, d1)

    base = jnp.minimum(sid * CH, H - CH)
    skip = sid * CH - base
    pb = pl.multiple_of(jnp.maximum(base - 16, 0), 8)
    base = pl.multiple_of(base, 8)
    pend_h = []
    for hi, hist_h in enumerate((srch_h, dsth_h)):
        pend_h.append((
            pltpu.async_copy(hist_h.at[pl.ds(pb, 16)], hb[hi].at[pl.ds(0, 16)], sem_h),
            pltpu.async_copy(hist_h.at[pl.ds(base, CH)], hb[hi].at[pl.ds(16, CH)], sem_h),
        ))
    pend_in = start_in(0, 0)

    zslice = (2 * WT) // NS
    for i in range(zslice // 16):
        stv[pl.ds(i * 16, 16)] = zero16
    pltpu.sync_copy(stv.at[pl.ds(0, zslice)], tbl_sh.at[pl.ds(sid * zslice, zslice)])

    plsc.subcore_barrier()

    for hi in range(2):
        for d in pend_h[hi]:
            d.wait()
        hbuf = hb[hi]
        toff = hi * WT

        stc[pl.ds(0, 16)] = zero16
        stj[pl.ds(0, 16)] = zero16 - 1

        @plsc.parallel_loop(0, CH_ROWS, step=1, unroll=4, carry=jnp.int32(8))
        def p1(jj, off, hbuf=hbuf, toff=toff):
            o = jj * 16
            v = hbuf[pl.ds(16 + o, 16)]
            prevv = hbuf[pl.ds(15 + o, 16)]
            nxtv = hbuf[pl.ds(17 + o, 16)]
            keep = (v > prevv) | ((base == 0) & (jj == 0) & (lanes == 0))
            keep = keep & ((o + lanes) >= skip)
            bit = jnp.where(keep, jnp.int32(1) << (v & 31), 0)
            w5 = v >> 5
            isend = (w5 != (nxtv >> 5)) | (lanes == 15)
            cs = plsc.cumsum(bit)
            jjv = jnp.zeros((16,), jnp.int32) + jj
            plsc.store_compressed(stw.at[pl.ds(off, 16)], w5 + toff, mask=isend)
            plsc.store_compressed(stc.at[pl.ds(off, 16)], cs, mask=isend)
            plsc.store_compressed(stj.at[pl.ds(off, 16)], jjv, mask=isend)
            pc = plsc.all_reduce_population_count(isend)
            return off + pc[0]

        cnt = p1 - 8

        def p2(kk, _):
            q = kk * 16
            cc = stc[pl.ds(8 + q, 16)]
            cp = stc[pl.ds(7 + q, 16)]
            jc = stj[pl.ds(8 + q, 16)]
            jp = stj[pl.ds(7 + q, 16)]
            stv[pl.ds(8 + q, 16)] = cc - jnp.where(jc != jp, 0, cp)
            return _

        lax.fori_loop(0, (cnt + 15) >> 4, p2, 0)

        for k in range(8):
            stw[pl.ds(cnt + 8 + k * 16, 16)] = zero16
            stv[pl.ds(cnt + 8 + k * 16, 16)] = zero16

        for r in range(ST2_ROWS):
            @pl.when(r * 128 < cnt)
            def _(r=r):
                for k in range(8):
                    stw2d[r, pl.ds(k * 16, 16)] = stw[pl.ds(8 + r * 128 + k * 16, 16)]
                pltpu.sync_copy(stv.at[pl.ds(8 + r * 128, 128)],
                                tbl_sh.at[stw2d.at[r]], add=True)

    plsc.subcore_barrier()

    dbs = pltpu.async_copy(tbl_sh.at[pl.ds(0, WT)], tbl_vs, sem_h)
    dbd = pltpu.async_copy(tbl_sh.at[pl.ds(WT, WT)], tbl_vd, sem_h)
    dbs.wait()
    dbd.wait()

    pend_out = [None, None]
    for ch in range(n_ch):
        b = ch & 1
        for d in pend_in:
            d.wait()
        if ch + 1 < n_ch:
            pend_in = start_in(ch + 1, 1 - b)
        if pend_out[b] is not None:
            pend_out[b].wait()
        es, ed, eo = ev_s[b], ev_d[b], ev_o[b]

        @plsc.parallel_loop(0, EC, step=16, unroll=16)
        def ebody(o):
            sv = es[pl.ds(o, 16)]
            dv = ed[pl.ds(o, 16)]
            sw = plsc.load_gather(tbl_vs, [sv >> 5])
            dw = plsc.load_gather(tbl_vd, [dv >> 5])
            hit = (sw >> (sv & 31)) & (dw >> (dv & 31)) & 1
            eo[pl.ds(o, 16)] = hit.astype(jnp.float32)

        cb = pl.multiple_of(ebase + ch * EC, 8)
        pend_out[b] = pltpu.async_copy(eo, out_h.at[pl.ds(cb, EC)], sem_out[b])
    for d in pend_out:
        if d is not None:
            d.wait()


@jax.jit
def _run(src, dst, src_hist, dst_hist):
    mesh = plsc.VectorSubcoreMesh(
        core_axis_name="c", subcore_axis_name="s", num_cores=NC, num_subcores=NS)
    k = pl.kernel(
        _body,
        out_type=jax.ShapeDtypeStruct((B,), jnp.float32),
        mesh=mesh,
        compiler_params=pltpu.CompilerParams(needs_layout_passes=False),
        scratch_types=[
            pltpu.VMEM_SHARED((2 * WT,), jnp.int32),
            [pltpu.VMEM((16 + CH + 16,), jnp.int32) for _ in range(2)],
            pltpu.VMEM((ST,), jnp.int32),
            pltpu.VMEM((ST,), jnp.int32),
            pltpu.VMEM((ST,), jnp.int32),
            pltpu.VMEM((ST,), jnp.int32),
            pltpu.VMEM((ST2_ROWS, 128), jnp.int32),
            pltpu.VMEM((WT,), jnp.int32),
            pltpu.VMEM((WT,), jnp.int32),
            pltpu.VMEM((EC,), jnp.int32),
            pltpu.VMEM((EC,), jnp.int32),
            pltpu.VMEM((EC,), jnp.float32),
            pltpu.VMEM((EC,), jnp.int32),
            pltpu.VMEM((EC,), jnp.int32),
            pltpu.VMEM((EC,), jnp.float32),
            pltpu.SemaphoreType.DMA,
            [pltpu.SemaphoreType.DMA, pltpu.SemaphoreType.DMA],
            [pltpu.SemaphoreType.DMA, pltpu.SemaphoreType.DMA],
        ],
    )
    return k(src, dst, src_hist, dst_hist)


def kernel(src, dst, t, msg, src_hist, dst_hist):
    return _run(src, dst, src_hist, dst_hist)
